# baseline (speedup 1.0000x reference)
import functools

import jax
import jax.numpy as jnp
from jax import lax
from jax.experimental import pallas as pl
from jax.experimental.pallas import tpu as pltpu
from jax.experimental.pallas import tpu_sc as plsc

D = 64
LANES = 16
E = 128
NCORES = 2
NSUB = 16
C = 25088
NCHUNK = 4
NPADROWS = NCHUNK * C
ROWS_PER_TILE = C // NSUB
ZROWS = 32
REG = 1e-4
N_LAYERS = 3

_mesh = plsc.VectorSubcoreMesh(core_axis_name="c", subcore_axis_name="s")


def _make_prop(nnz_pad):
    per_tile = nnz_pad // NSUB
    nblk = per_tile // E

    @functools.partial(
        pl.kernel,
        out_type=jax.ShapeDtypeStruct((NPADROWS, D), jnp.float32),
        mesh=_mesh,
        scratch_types=[
            pltpu.VMEM_SHARED((C, D), jnp.float32),
            pltpu.VMEM((E,), jnp.int32),
            pltpu.VMEM((E,), jnp.int32),
            pltpu.VMEM((E,), jnp.float32),
            pltpu.VMEM((E,), jnp.float32),
            pltpu.VMEM((E,), jnp.int32),
            pltpu.VMEM((E, D), jnp.float32),
            pltpu.VMEM((ZROWS, D), jnp.float32),
            pltpu.SemaphoreType.DMA,
        ],
        compiler_params=pltpu.CompilerParams(use_tc_tiling_on_sc=False),
    )
    def prop(x_hbm, src_hbm, dst_hbm, w_hbm, out_hbm,
             acc, src_b, dst_b, w_b, wm_b, dloc_b, rows_b, zero_b, sem):
        cid = lax.axis_index("c")
        sid = lax.axis_index("s")

        @pl.loop(0, ZROWS)
        def _zinit(r):
            for c4 in range(D // LANES):
                zero_b[r, pl.ds(c4 * LANES, LANES)] = jnp.zeros(
                    (LANES,), jnp.float32)

        for chunk_i in range(NCHUNK // NCORES):
            chunk = cid * (NCHUNK // NCORES) + chunk_i
            base = chunk * C

            @pl.loop(0, ROWS_PER_TILE // ZROWS)
            def _zero(i):
                row0 = sid * ROWS_PER_TILE + i * ZROWS
                pltpu.sync_copy(zero_b, acc.at[pl.ds(row0, ZROWS)])

            plsc.subcore_barrier()

            @pl.loop(0, nblk)
            def _blk(blk):
                e0 = pl.multiple_of(sid * per_tile + blk * E, E)
                pltpu.sync_copy(src_hbm.at[pl.ds(e0, E)], src_b)
                pltpu.sync_copy(dst_hbm.at[pl.ds(e0, E)], dst_b)
                pltpu.sync_copy(w_hbm.at[pl.ds(e0, E)], w_b)
                pltpu.async_copy(x_hbm.at[src_b], rows_b, sem).wait()
                for v in range(E // LANES):
                    sl = pl.ds(v * LANES, LANES)
                    d = dst_b[sl]
                    inm = (d >= base) & (d < base + C)
                    wm_b[sl] = jnp.where(inm, w_b[sl], jnp.float32(0.0))
                    dloc_b[sl] = jnp.where(inm, d - base, 0)

                @pl.loop(0, E // LANES)
                def _scale(g):
                    wv = wm_b[pl.ds(g * LANES, LANES)]
                    for lane in range(LANES):
                        r = g * LANES + lane
                        ww = wv[lane]
                        for c4 in range(D // LANES):
                            cs = pl.ds(c4 * LANES, LANES)
                            rows_b[r, cs] = rows_b[r, cs] * ww

                pltpu.sync_copy(rows_b, acc.at[dloc_b], add=True)

            plsc.subcore_barrier()

            row0 = sid * ROWS_PER_TILE
            pltpu.sync_copy(acc.at[pl.ds(row0, ROWS_PER_TILE)],
                            out_hbm.at[pl.ds(base + row0, ROWS_PER_TILE)])

    return prop


def _make_gather_mean(b3):
    rows_per_w = b3 // (NCORES * NSUB)
    nblk = rows_per_w // E

    @functools.partial(
        pl.kernel,
        out_type=jax.ShapeDtypeStruct((b3, D), jnp.float32),
        mesh=_mesh,
        scratch_types=[
            pltpu.VMEM((E,), jnp.int32),
            pltpu.VMEM((E, D), jnp.float32),
            pltpu.VMEM((E, D), jnp.float32),
            pltpu.SemaphoreType.DMA,
        ],
        compiler_params=pltpu.CompilerParams(use_tc_tiling_on_sc=False),
    )
    def gmean(x0, x1, x2, x3, idx_hbm, out_hbm, idx_b, rows_b, acc_b, sem):
        cid = lax.axis_index("c")
        sid = lax.axis_index("s")
        wid = sid * NCORES + cid
        for blk in range(nblk):
            r0 = pl.multiple_of(wid * rows_per_w + blk * E, E)
            pltpu.sync_copy(idx_hbm.at[pl.ds(r0, E)], idx_b)
            pltpu.async_copy(x0.at[idx_b], acc_b, sem).wait()
            for xl in (x1, x2, x3):
                pltpu.async_copy(xl.at[idx_b], rows_b, sem).wait()

                @pl.loop(0, E)
                def _add(r):
                    for c4 in range(D // LANES):
                        cs = pl.ds(c4 * LANES, LANES)
                        acc_b[r, cs] = acc_b[r, cs] + rows_b[r, cs]

            @pl.loop(0, E)
            def _mean(r):
                for c4 in range(D // LANES):
                    cs = pl.ds(c4 * LANES, LANES)
                    acc_b[r, cs] = acc_b[r, cs] * jnp.float32(0.25)

            pltpu.sync_copy(acc_b, out_hbm.at[pl.ds(r0, E)])

    return gmean


def _loss_body(u_ref, p_ref, n_ref, o_ref):
    u = u_ref[...]
    p = p_ref[...]
    n = n_ref[...]
    pos_scores = jnp.sum(u * p, axis=1)
    neg_scores = jnp.sum(u * n, axis=1)
    loss = -jnp.mean(jax.nn.log_sigmoid(pos_scores - neg_scores))
    batch = u.shape[0]
    reg_term = jnp.sum(u * u) + jnp.sum(p * p) + jnp.sum(n * n)
    o_ref[0, 0] = loss + REG * reg_term / batch


def kernel(user_emb, item_emb, edge_index, edge_weight, users, pos_items,
           neg_items):
    n_users = user_emb.shape[0]
    n_nodes = n_users + item_emb.shape[0]
    nnz = edge_index.shape[1]
    batch = users.shape[0]

    nnz_pad = -(-nnz // (NSUB * E)) * (NSUB * E)

    x0 = jnp.concatenate([user_emb, item_emb], axis=0)
    x0 = jnp.pad(x0, ((0, NPADROWS - n_nodes), (0, 0)))
    dst = jnp.pad(edge_index[0], (0, nnz_pad - nnz))
    src = jnp.pad(edge_index[1], (0, nnz_pad - nnz))
    w = jnp.pad(edge_weight, (0, nnz_pad - nnz))

    prop = _make_prop(nnz_pad)
    xs = [x0]
    for _ in range(N_LAYERS):
        xs.append(prop(xs[-1], src, dst, w))

    idx_all = jnp.concatenate(
        [users, n_users + pos_items, n_users + neg_items]).astype(jnp.int32)
    gmean = _make_gather_mean(idx_all.shape[0])
    rows = gmean(xs[0], xs[1], xs[2], xs[3], idx_all)

    u = rows[:batch]
    p = rows[batch:2 * batch]
    n = rows[2 * batch:]

    out = pl.pallas_call(
        _loss_body,
        out_shape=jax.ShapeDtypeStruct((1, 1), jnp.float32),
        in_specs=[pl.BlockSpec(memory_space=pltpu.VMEM)] * 3,
        out_specs=pl.BlockSpec(memory_space=pltpu.SMEM),
    )(u, p, n)
    return out[0, 0]


# speedup vs baseline: 1.5230x; 1.5230x over previous
import functools

import jax
import jax.numpy as jnp
from jax import lax
from jax.experimental import pallas as pl
from jax.experimental.pallas import tpu as pltpu
from jax.experimental.pallas import tpu_sc as plsc

D = 64
LANES = 16
E = 128
NCORES = 2
NSUB = 16
C = 25088
NCHUNK = 4
NPADROWS = NCHUNK * C
ROWS_PER_TILE = C // NSUB
ZROWS = 32
REG = 1e-4
N_LAYERS = 3

_mesh = plsc.VectorSubcoreMesh(core_axis_name="c", subcore_axis_name="s")


SUPER = 8
ME = SUPER * E


def _make_prop(nnz_pad):
    per_tile = nnz_pad // NSUB
    nsup = per_tile // ME
    assert nsup % 2 == 0

    @functools.partial(
        pl.kernel,
        out_type=jax.ShapeDtypeStruct((NPADROWS, D), jnp.float32),
        mesh=_mesh,
        scratch_types=[
            pltpu.VMEM_SHARED((C, D), jnp.float32),
            pltpu.VMEM((ME,), jnp.int32),
            pltpu.VMEM((ME,), jnp.int32),
            pltpu.VMEM((ME,), jnp.float32),
            pltpu.VMEM((ME,), jnp.int32),
            pltpu.VMEM((ME,), jnp.int32),
            pltpu.VMEM((ME,), jnp.float32),
            pltpu.VMEM((E,), jnp.float32),
            pltpu.VMEM((E,), jnp.int32),
            pltpu.VMEM((E,), jnp.float32),
            pltpu.VMEM((E,), jnp.int32),
            pltpu.VMEM((E, D), jnp.float32),
            pltpu.VMEM((E, D), jnp.float32),
            pltpu.VMEM((ZROWS, D), jnp.float32),
            pltpu.SemaphoreType.DMA,
            pltpu.SemaphoreType.DMA,
            pltpu.SemaphoreType.DMA,
            pltpu.SemaphoreType.DMA,
        ],
        compiler_params=pltpu.CompilerParams(use_tc_tiling_on_sc=False),
    )
    def prop(x_hbm, src_hbm, dst_hbm, w_hbm, out_hbm,
             acc, src0, dst0, w0, src1, dst1, w1,
             wm0, dloc0, wm1, dloc1, rows0, rows1, zero_b,
             msem0, msem1, gsem0, gsem1):
        cid = lax.axis_index("c")
        sid = lax.axis_index("s")
        meta = ((src0, dst0, w0, msem0), (src1, dst1, w1, msem1))
        rbuf = ((rows0, gsem0), (rows1, gsem1))
        mask = ((wm0, dloc0), (wm1, dloc1))

        @pl.loop(0, ZROWS)
        def _zinit(r):
            for c4 in range(D // LANES):
                zero_b[r, pl.ds(c4 * LANES, LANES)] = jnp.zeros(
                    (LANES,), jnp.float32)

        def meta_start(sup, m):
            sb, db, wb, sem = meta[m]
            e0 = pl.multiple_of(sid * per_tile + sup * ME, ME)
            pltpu.async_copy(src_hbm.at[pl.ds(e0, ME)], sb, sem)
            pltpu.async_copy(dst_hbm.at[pl.ds(e0, ME)], db, sem)
            pltpu.async_copy(w_hbm.at[pl.ds(e0, ME)], wb, sem)

        def meta_wait(m):
            sb, db, wb, sem = meta[m]
            pltpu.make_async_copy(src_hbm.at[pl.ds(0, ME)], sb, sem).wait()
            pltpu.make_async_copy(dst_hbm.at[pl.ds(0, ME)], db, sem).wait()
            pltpu.make_async_copy(w_hbm.at[pl.ds(0, ME)], wb, sem).wait()

        def gather_start(m, b, r):
            sb = meta[m][0]
            rows, sem = rbuf[r]
            pltpu.async_copy(x_hbm.at[sb.at[pl.ds(b * E, E)]], rows, sem)

        def gather_wait(m, b, r):
            sb = meta[m][0]
            rows, sem = rbuf[r]
            pltpu.make_async_copy(
                x_hbm.at[sb.at[pl.ds(b * E, E)]], rows, sem).wait()

        for chunk_i in range(NCHUNK // NCORES):
            chunk = cid * (NCHUNK // NCORES) + chunk_i
            base = chunk * C

            @pl.loop(0, ROWS_PER_TILE // ZROWS)
            def _zero(i):
                row0 = sid * ROWS_PER_TILE + i * ZROWS
                pltpu.sync_copy(zero_b, acc.at[pl.ds(row0, ZROWS)])

            plsc.subcore_barrier()

            def process(m, b, r):
                _, db, wb, _ = meta[m]
                rows, _ = rbuf[r]
                wm, dloc = mask[r]
                gather_wait(m, b, r)
                for v in range(E // LANES):
                    sl = pl.ds(v * LANES, LANES)
                    d = db[pl.ds(b * E + v * LANES, LANES)]
                    inm = (d >= base) & (d < base + C)
                    wm[sl] = jnp.where(inm, wb[pl.ds(b * E + v * LANES,
                                                     LANES)],
                                       jnp.float32(0.0))
                    dloc[sl] = jnp.where(inm, d - base, 0)

                @pl.loop(0, E // LANES)
                def _scale(g):
                    wv = wm[pl.ds(g * LANES, LANES)]
                    for lane in range(LANES):
                        rr = g * LANES + lane
                        ww = wv[lane]
                        for c4 in range(D // LANES):
                            cs = pl.ds(c4 * LANES, LANES)
                            rows[rr, cs] = rows[rr, cs] * ww

                pltpu.sync_copy(rows, acc.at[dloc], add=True)

            def process_super(j, m):
                sup = 2 * j + m
                gather_start(m, 0, 0)

                @pl.loop(0, SUPER // 2)
                def _pair(k):
                    gather_start(m, 2 * k + 1, 1)
                    process(m, 2 * k, 0)

                    @pl.when(k < SUPER // 2 - 1)
                    def _pre():
                        gather_start(m, 2 * k + 2, 0)

                    process(m, 2 * k + 1, 1)

            meta_start(0, 0)

            @pl.loop(0, nsup // 2)
            def _sup(j):
                meta_wait(0)
                meta_start(2 * j + 1, 1)
                process_super(j, 0)
                meta_wait(1)

                @pl.when(j < nsup // 2 - 1)
                def _pre_meta():
                    meta_start(2 * j + 2, 0)

                process_super(j, 1)

            plsc.subcore_barrier()

            row0 = sid * ROWS_PER_TILE
            pltpu.sync_copy(acc.at[pl.ds(row0, ROWS_PER_TILE)],
                            out_hbm.at[pl.ds(base + row0, ROWS_PER_TILE)])

    return prop


def _make_gather_mean(b3):
    rows_per_w = b3 // (NCORES * NSUB)
    nblk = rows_per_w // E

    @functools.partial(
        pl.kernel,
        out_type=jax.ShapeDtypeStruct((b3, D), jnp.float32),
        mesh=_mesh,
        scratch_types=[
            pltpu.VMEM((E,), jnp.int32),
            pltpu.VMEM((E, D), jnp.float32),
            pltpu.VMEM((E, D), jnp.float32),
            pltpu.SemaphoreType.DMA,
        ],
        compiler_params=pltpu.CompilerParams(use_tc_tiling_on_sc=False),
    )
    def gmean(x0, x1, x2, x3, idx_hbm, out_hbm, idx_b, rows_b, acc_b, sem):
        cid = lax.axis_index("c")
        sid = lax.axis_index("s")
        wid = sid * NCORES + cid
        for blk in range(nblk):
            r0 = pl.multiple_of(wid * rows_per_w + blk * E, E)
            pltpu.sync_copy(idx_hbm.at[pl.ds(r0, E)], idx_b)
            pltpu.async_copy(x0.at[idx_b], acc_b, sem).wait()
            for xl in (x1, x2, x3):
                pltpu.async_copy(xl.at[idx_b], rows_b, sem).wait()

                @pl.loop(0, E)
                def _add(r):
                    for c4 in range(D // LANES):
                        cs = pl.ds(c4 * LANES, LANES)
                        acc_b[r, cs] = acc_b[r, cs] + rows_b[r, cs]

            @pl.loop(0, E)
            def _mean(r):
                for c4 in range(D // LANES):
                    cs = pl.ds(c4 * LANES, LANES)
                    acc_b[r, cs] = acc_b[r, cs] * jnp.float32(0.25)

            pltpu.sync_copy(acc_b, out_hbm.at[pl.ds(r0, E)])

    return gmean


def _loss_body(u_ref, p_ref, n_ref, o_ref):
    u = u_ref[...]
    p = p_ref[...]
    n = n_ref[...]
    pos_scores = jnp.sum(u * p, axis=1)
    neg_scores = jnp.sum(u * n, axis=1)
    loss = -jnp.mean(jax.nn.log_sigmoid(pos_scores - neg_scores))
    batch = u.shape[0]
    reg_term = jnp.sum(u * u) + jnp.sum(p * p) + jnp.sum(n * n)
    o_ref[0, 0] = loss + REG * reg_term / batch


def kernel(user_emb, item_emb, edge_index, edge_weight, users, pos_items,
           neg_items):
    n_users = user_emb.shape[0]
    n_nodes = n_users + item_emb.shape[0]
    nnz = edge_index.shape[1]
    batch = users.shape[0]

    nnz_pad = -(-nnz // (NSUB * 2 * ME)) * (NSUB * 2 * ME)

    x0 = jnp.concatenate([user_emb, item_emb], axis=0)
    x0 = jnp.pad(x0, ((0, NPADROWS - n_nodes), (0, 0)))
    dst = jnp.pad(edge_index[0], (0, nnz_pad - nnz))
    src = jnp.pad(edge_index[1], (0, nnz_pad - nnz))
    w = jnp.pad(edge_weight, (0, nnz_pad - nnz))

    prop = _make_prop(nnz_pad)
    xs = [x0]
    for _ in range(N_LAYERS):
        xs.append(prop(xs[-1], src, dst, w))

    idx_all = jnp.concatenate(
        [users, n_users + pos_items, n_users + neg_items]).astype(jnp.int32)
    gmean = _make_gather_mean(idx_all.shape[0])
    rows = gmean(xs[0], xs[1], xs[2], xs[3], idx_all)

    u = rows[:batch]
    p = rows[batch:2 * batch]
    n = rows[2 * batch:]

    out = pl.pallas_call(
        _loss_body,
        out_shape=jax.ShapeDtypeStruct((1, 1), jnp.float32),
        in_specs=[pl.BlockSpec(memory_space=pltpu.VMEM)] * 3,
        out_specs=pl.BlockSpec(memory_space=pltpu.SMEM),
    )(u, p, n)
    return out[0, 0]


# speedup vs baseline: 4.6995x; 3.0858x over previous
import functools

import jax
import jax.numpy as jnp
from jax import lax
from jax.experimental import pallas as pl
from jax.experimental.pallas import tpu as pltpu
from jax.experimental.pallas import tpu_sc as plsc

D = 64
LANES = 16
E = 128
NCORES = 2
NSUB = 16
NW = NCORES * NSUB
C = 25088
NCHUNK = 4
NPADROWS = NCHUNK * C
ROWS_PER_TILE = C // NSUB
ZROWS = 32
REG = 1e-4
N_LAYERS = 3
ME = 1024
RW = 16

_mesh = plsc.VectorSubcoreMesh(core_axis_name="c", subcore_axis_name="s")
_params = pltpu.CompilerParams(use_tc_tiling_on_sc=False,
                               needs_layout_passes=False)


def _iota():
    return lax.iota(jnp.int32, LANES)


def _chunk_id(d):
    kv = jnp.zeros(d.shape, jnp.int32)
    for k in range(1, NCHUNK):
        kv = kv + jnp.where(d >= k * C, 1, 0).astype(jnp.int32)
    return kv


def _make_count(nnz_pad):
    per_w = nnz_pad // NW
    nsup = per_w // ME

    @functools.partial(
        pl.kernel,
        out_type=jax.ShapeDtypeStruct((NW, LANES), jnp.int32),
        mesh=_mesh,
        scratch_types=[
            pltpu.VMEM((ME,), jnp.int32),
            pltpu.VMEM((LANES,), jnp.int32),
        ],
        compiler_params=_params,
    )
    def count(dst_hbm, counts_out, dbuf, cbuf):
        cid = lax.axis_index("c")
        sid = lax.axis_index("s")
        wid = sid * NCORES + cid
        e_base = wid * per_w

        @pl.loop(0, nsup, init_carry=(jnp.int32(0),) * NCHUNK)
        def totals(s, car):
            e0 = pl.multiple_of(e_base + s * ME, ME)
            pltpu.sync_copy(dst_hbm.at[pl.ds(e0, ME)], dbuf)

            @pl.loop(0, ME // LANES, init_carry=car)
            def car2(v, c2):
                d = dbuf[pl.ds(v * LANES, LANES)]
                kv = _chunk_id(d)
                out = []
                for k in range(NCHUNK):
                    cnt = plsc.all_reduce_population_count(kv == k)[0]
                    out.append(c2[k] + cnt)
                return tuple(out)

            return car2

        vec = jnp.zeros((LANES,), jnp.int32)
        ii = _iota()
        for k in range(NCHUNK):
            vec = jnp.where(ii == k, totals[k], vec)
        cbuf[pl.ds(0, LANES)] = vec
        pltpu.sync_copy(cbuf, counts_out.at[wid])

    return count


def _make_scatter(nnz_pad, nrec):
    per_w = nnz_pad // NW
    nvec = per_w // LANES
    vps = ME // LANES
    trash = nnz_pad + NCHUNK * E

    @functools.partial(
        pl.kernel,
        out_type=(jax.ShapeDtypeStruct((nrec, RW), jnp.float32),
                  jax.ShapeDtypeStruct((2, LANES), jnp.int32)),
        mesh=_mesh,
        scratch_types=[
            pltpu.VMEM((ME,), jnp.int32),
            pltpu.VMEM((ME,), jnp.int32),
            pltpu.VMEM((ME,), jnp.float32),
            pltpu.VMEM((NW, LANES), jnp.int32),
            pltpu.VMEM((LANES, RW), jnp.float32),
            pltpu.VMEM((LANES, RW), jnp.float32),
            pltpu.VMEM((LANES, RW), jnp.float32),
            pltpu.VMEM((LANES,), jnp.int32),
            pltpu.VMEM((LANES,), jnp.int32),
            pltpu.VMEM((LANES,), jnp.int32),
            pltpu.VMEM((LANES,), jnp.int32),
            pltpu.SemaphoreType.DMA,
            pltpu.SemaphoreType.DMA,
            pltpu.SemaphoreType.DMA,
        ],
        compiler_params=_params,
    )
    def scatter(counts_hbm, src_hbm, dst_hbm, w_hbm, rec_out, meta_out,
                sbuf, dbuf, wbuf, cnts_v, stage0, stage1, zstage, mbuf,
                pidx0, pidx1, tidx, sem0, sem1, tsem):
        cid = lax.axis_index("c")
        sid = lax.axis_index("s")
        wid = sid * NCORES + cid
        ii = _iota()
        zi = jnp.zeros((LANES,), jnp.int32)

        pltpu.sync_copy(counts_hbm, cnts_v)

        @pl.loop(0, NW, init_carry=jnp.zeros((LANES,), jnp.int32))
        def tot(i, t):
            return t + cnts_v[i]

        pcap = ((tot + (E - 1)) // E) * E
        base_v = plsc.cumsum(pcap) - pcap

        @pl.loop(0, wid, init_carry=jnp.zeros((LANES,), jnp.int32))
        def own(i, t):
            return t + cnts_v[i]

        wcur_v = base_v + own
        wcur = tuple(wcur_v[k] for k in range(NCHUNK))

        @pl.when(wid < NCHUNK)
        def _tail():
            @pl.loop(0, LANES)
            def _z(r):
                zstage[r, pl.ds(0, LANES)] = jnp.zeros((LANES,),
                                                       jnp.float32)

            def sel(vec):
                s = vec[0]
                for k in range(1, NCHUNK):
                    s = jnp.where(wid == k, vec[k], s)
                return s

            ts = sel(base_v) + sel(tot)
            te = sel(base_v) + sel(pcap)
            for t in range(E // LANES):
                pos = ts + t * LANES + ii
                posm = jnp.where(pos < te, pos, trash + ii)
                tidx[pl.ds(0, LANES)] = posm
                pltpu.async_copy(zstage, rec_out.at[tidx], tsem).wait()

        def e_base_for(v0):
            return wid * per_w + (v0 // vps) * ME

        def emit(v, stage, pidx, sem, car):
            off = (v % vps) * LANES
            d = dbuf[pl.ds(off, LANES)]
            s_ = sbuf[pl.ds(off, LANES)]
            w_ = wbuf[pl.ds(off, LANES)]
            kv = _chunk_id(d)
            dloc = d - kv * C
            pos = jnp.zeros((LANES,), jnp.int32)
            ncar = []
            for k in range(NCHUNK):
                m = kv == k
                pref = plsc.cumsum(jnp.where(m, 1, 0))
                pos = jnp.where(m, car[k] + pref - 1, pos)
                ncar.append(car[k] + pref[LANES - 1])
            plsc.store_scatter(stage, [ii, zi], plsc.bitcast(s_,
                                                             jnp.float32))
            plsc.store_scatter(stage, [ii, zi + 1],
                               plsc.bitcast(dloc, jnp.float32))
            plsc.store_scatter(stage, [ii, zi + 2], w_)
            pidx[pl.ds(0, LANES)] = pos
            pltpu.async_copy(stage, rec_out.at[pidx], sem)
            return tuple(ncar)

        @pl.loop(0, nvec // 2, init_carry=wcur)
        def _main(v2, car):
            v0 = 2 * v2

            @pl.when(v0 % vps == 0)
            def _load():
                e0 = pl.multiple_of(e_base_for(v0), ME)
                pltpu.sync_copy(src_hbm.at[pl.ds(e0, ME)], sbuf)
                pltpu.sync_copy(dst_hbm.at[pl.ds(e0, ME)], dbuf)
                pltpu.sync_copy(w_hbm.at[pl.ds(e0, ME)], wbuf)

            @pl.when(v2 > 0)
            def _drain():
                pltpu.make_async_copy(stage0, rec_out.at[pidx0],
                                      sem0).wait()
                pltpu.make_async_copy(stage1, rec_out.at[pidx1],
                                      sem1).wait()

            car = emit(v0, stage0, pidx0, sem0, car)
            car = emit(v0 + 1, stage1, pidx1, sem1, car)
            return car

        pltpu.make_async_copy(stage0, rec_out.at[pidx0], sem0).wait()
        pltpu.make_async_copy(stage1, rec_out.at[pidx1], sem1).wait()

        @pl.when(wid == 0)
        def _meta():
            mbuf[pl.ds(0, LANES)] = base_v
            pltpu.sync_copy(mbuf, meta_out.at[0])
            mbuf[pl.ds(0, LANES)] = pcap // E
            pltpu.sync_copy(mbuf, meta_out.at[1])

    return scatter


def _make_prop(nrec):
    @functools.partial(
        pl.kernel,
        out_type=jax.ShapeDtypeStruct((NPADROWS, D), jnp.float32),
        mesh=_mesh,
        scratch_types=[
            pltpu.VMEM_SHARED((C, D), jnp.float32),
            pltpu.VMEM((2, LANES), jnp.int32),
            pltpu.VMEM((E, RW), jnp.float32),
            pltpu.VMEM((E, RW), jnp.float32),
            pltpu.VMEM((E,), jnp.int32),
            pltpu.VMEM((E,), jnp.int32),
            pltpu.VMEM((E,), jnp.int32),
            pltpu.VMEM((E,), jnp.int32),
            pltpu.VMEM((E,), jnp.float32),
            pltpu.VMEM((E,), jnp.float32),
            pltpu.VMEM((E, D), jnp.float32),
            pltpu.VMEM((E, D), jnp.float32),
            pltpu.VMEM((ZROWS, D), jnp.float32),
            pltpu.SemaphoreType.DMA,
            pltpu.SemaphoreType.DMA,
            pltpu.SemaphoreType.DMA,
            pltpu.SemaphoreType.DMA,
        ],
        compiler_params=_params,
    )
    def prop(x_hbm, rec_hbm, meta_hbm, out_hbm,
             acc, meta_v, rec0, rec1, s0, s1, dl0, dl1, w0, w1,
             rows0, rows1, zero_b, rsem0, rsem1, gsem0, gsem1):
        cid = lax.axis_index("c")
        sid = lax.axis_index("s")
        ii = _iota()
        recb = ((rec0, rsem0), (rec1, rsem1))
        cols = ((s0, dl0, w0), (s1, dl1, w1))
        rowb = ((rows0, gsem0), (rows1, gsem1))

        pltpu.sync_copy(meta_hbm, meta_v)
        base_vec = meta_v[0]
        nblk_vec = meta_v[1]

        @pl.loop(0, ZROWS)
        def _zinit(r):
            for c4 in range(D // LANES):
                zero_b[r, pl.ds(c4 * LANES, LANES)] = jnp.zeros(
                    (LANES,), jnp.float32)

        def rec_start(bb, b, p):
            buf, sem = recb[p]
            row0 = pl.multiple_of(bb + b * E, E)
            pltpu.async_copy(rec_hbm.at[pl.ds(row0, E)], buf, sem)

        def rec_wait(p):
            buf, sem = recb[p]
            pltpu.make_async_copy(rec_hbm.at[pl.ds(0, E)], buf, sem).wait()

        def extract_and_gather(p):
            buf, _ = recb[p]
            sb, dlb, wb = cols[p]
            rows, gsem = rowb[p]
            for g in range(E // LANES):
                ids = ii + g * LANES
                sl = pl.ds(g * LANES, LANES)
                sb[sl] = plsc.bitcast(
                    plsc.load_gather(buf, [ids, jnp.zeros((LANES,),
                                                          jnp.int32)]),
                    jnp.int32)
                dlb[sl] = plsc.bitcast(
                    plsc.load_gather(buf, [ids, jnp.zeros((LANES,),
                                                          jnp.int32) + 1]),
                    jnp.int32)
                wb[sl] = plsc.load_gather(buf,
                                          [ids, jnp.zeros((LANES,),
                                                          jnp.int32) + 2])
            pltpu.async_copy(x_hbm.at[sb], rows, gsem)

        def process(p):
            sb, dlb, wb = cols[p]
            rows, gsem = rowb[p]
            pltpu.make_async_copy(x_hbm.at[sb], rows, gsem).wait()

            @pl.loop(0, E // LANES)
            def _scale(g):
                wv = wb[pl.ds(g * LANES, LANES)]
                for lane in range(LANES):
                    rr = g * LANES + lane
                    ww = wv[lane]
                    for c4 in range(D // LANES):
                        cs = pl.ds(c4 * LANES, LANES)
                        rows[rr, cs] = rows[rr, cs] * ww

            pltpu.sync_copy(rows, acc.at[dlb], add=True)

        for chunk_i in range(NCHUNK // NCORES):
            chunk = cid * (NCHUNK // NCORES) + chunk_i
            base_rows = chunk * C

            def sel(vec):
                s = vec[0]
                for k in range(1, NCHUNK):
                    s = jnp.where(chunk == k, vec[k], s)
                return s

            bb = sel(base_vec)
            nb = sel(nblk_vec)

            @pl.loop(0, ROWS_PER_TILE // ZROWS)
            def _zero(i):
                row0 = sid * ROWS_PER_TILE + i * ZROWS
                pltpu.sync_copy(zero_b, acc.at[pl.ds(row0, ZROWS)])

            plsc.subcore_barrier()

            ntile = (nb - sid + NSUB - 1) // NSUB

            def blk(i):
                return (sid + i * NSUB)

            @pl.when(ntile > 0)
            def _prologue():
                rec_start(bb, blk(0), 0)
                rec_wait(0)
                extract_and_gather(0)

                @pl.when(ntile > 1)
                def _p2():
                    rec_start(bb, blk(1), 1)

            @pl.loop(0, (ntile + 1) // 2)
            def _pair(k):
                i0 = 2 * k
                i1 = i0 + 1

                @pl.when(i1 < ntile)
                def _odd_front():
                    rec_wait(1)
                    extract_and_gather(1)

                @pl.when(i0 + 2 < ntile)
                def _pre0():
                    rec_start(bb, blk(i0 + 2), 0)

                process(0)

                @pl.when(i1 < ntile)
                def _odd_back():
                    @pl.when(i1 + 2 < ntile)
                    def _pre1():
                        rec_start(bb, blk(i1 + 2), 1)

                    @pl.when(i0 + 2 < ntile)
                    def _next0():
                        rec_wait(0)
                        extract_and_gather(0)

                    process(1)

            plsc.subcore_barrier()

            row0 = sid * ROWS_PER_TILE
            pltpu.sync_copy(acc.at[pl.ds(row0, ROWS_PER_TILE)],
                            out_hbm.at[pl.ds(base_rows + row0,
                                             ROWS_PER_TILE)])

    return prop


def _make_gather_mean(b3):
    rows_per_w = b3 // NW
    nblk = rows_per_w // E

    @functools.partial(
        pl.kernel,
        out_type=jax.ShapeDtypeStruct((b3, D), jnp.float32),
        mesh=_mesh,
        scratch_types=[
            pltpu.VMEM((E,), jnp.int32),
            pltpu.VMEM((E, D), jnp.float32),
            pltpu.VMEM((E, D), jnp.float32),
            pltpu.SemaphoreType.DMA,
        ],
        compiler_params=_params,
    )
    def gmean(x0, x1, x2, x3, idx_hbm, out_hbm, idx_b, rows_b, acc_b, sem):
        cid = lax.axis_index("c")
        sid = lax.axis_index("s")
        wid = sid * NCORES + cid
        for blk in range(nblk):
            r0 = pl.multiple_of(wid * rows_per_w + blk * E, E)
            pltpu.sync_copy(idx_hbm.at[pl.ds(r0, E)], idx_b)
            pltpu.async_copy(x0.at[idx_b], acc_b, sem).wait()
            for xl in (x1, x2, x3):
                pltpu.async_copy(xl.at[idx_b], rows_b, sem).wait()

                @pl.loop(0, E)
                def _add(r):
                    for c4 in range(D // LANES):
                        cs = pl.ds(c4 * LANES, LANES)
                        acc_b[r, cs] = acc_b[r, cs] + rows_b[r, cs]

            @pl.loop(0, E)
            def _mean(r):
                for c4 in range(D // LANES):
                    cs = pl.ds(c4 * LANES, LANES)
                    acc_b[r, cs] = acc_b[r, cs] * jnp.float32(0.25)

            pltpu.sync_copy(acc_b, out_hbm.at[pl.ds(r0, E)])

    return gmean


def _loss_body(u_ref, p_ref, n_ref, o_ref):
    u = u_ref[...]
    p = p_ref[...]
    n = n_ref[...]
    pos_scores = jnp.sum(u * p, axis=1)
    neg_scores = jnp.sum(u * n, axis=1)
    loss = -jnp.mean(jax.nn.log_sigmoid(pos_scores - neg_scores))
    batch = u.shape[0]
    reg_term = jnp.sum(u * u) + jnp.sum(p * p) + jnp.sum(n * n)
    o_ref[0, 0] = loss + REG * reg_term / batch


def kernel(user_emb, item_emb, edge_index, edge_weight, users, pos_items,
           neg_items):
    n_users = user_emb.shape[0]
    n_nodes = n_users + item_emb.shape[0]
    nnz = edge_index.shape[1]
    batch = users.shape[0]

    nnz_pad = -(-nnz // (NW * ME)) * (NW * ME)
    nrec = nnz_pad + NCHUNK * E + LANES

    x0 = jnp.concatenate([user_emb, item_emb], axis=0)
    x0 = jnp.pad(x0, ((0, NPADROWS - n_nodes), (0, 0)))
    dst = jnp.pad(edge_index[0], (0, nnz_pad - nnz))
    src = jnp.pad(edge_index[1], (0, nnz_pad - nnz))
    w = jnp.pad(edge_weight, (0, nnz_pad - nnz))

    counts = _make_count(nnz_pad)(dst)
    rec, meta = _make_scatter(nnz_pad, nrec)(counts, src, dst, w)

    prop = _make_prop(nrec)
    xs = [x0]
    for _ in range(N_LAYERS):
        xs.append(prop(xs[-1], rec, meta))

    idx_all = jnp.concatenate(
        [users, n_users + pos_items, n_users + neg_items]).astype(jnp.int32)
    gmean = _make_gather_mean(idx_all.shape[0])
    rows = gmean(xs[0], xs[1], xs[2], xs[3], idx_all)

    u = rows[:batch]
    p = rows[batch:2 * batch]
    n = rows[2 * batch:]

    out = pl.pallas_call(
        _loss_body,
        out_shape=jax.ShapeDtypeStruct((1, 1), jnp.float32),
        in_specs=[pl.BlockSpec(memory_space=pltpu.VMEM)] * 3,
        out_specs=pl.BlockSpec(memory_space=pltpu.SMEM),
    )(u, p, n)
    return out[0, 0]
